# baseline (speedup 1.0000x reference)
import functools

import jax
import jax.numpy as jnp
from jax import lax
from jax.experimental import pallas as pl
from jax.experimental.pallas import tpu as pltpu
from jax.experimental.pallas import tpu_sc as plsc

N = 10000
D = 128
E = 320000
NC = 2
NS = 16
NW = NC * NS
K = 128
G = 3
CHUNKS = 81
EPT = CHUNKS * K
EPAD = EPT * NW
ACC = 10112
RPT = ACC // NS
PADROW = 10048
DG = 8
DCH = 80
DEPAD = DCH * K * NW

_MESH = plsc.VectorSubcoreMesh(core_axis_name="c", subcore_axis_name="s")


def _zero_acc_slice(zrows, acc_sh, s):
    for b in range(4):
        pltpu.sync_copy(zrows, acc_sh.at[pl.ds(s * RPT + b * 128, 128)])
    pltpu.sync_copy(zrows.at[pl.ds(0, RPT - 512)],
                    acc_sh.at[pl.ds(s * RPT + 512, RPT - 512)])


def _deg_body(dst_hbm, out0, out1, dst_all, ones_v, acc_sh, sem):
    c = lax.axis_index("c")
    s = lax.axis_index("s")
    wid = s * NC + c
    z16 = jnp.zeros((16,), jnp.float32)
    one16 = jnp.ones((16,), jnp.float32)

    def zb(k, _):
        ones_v[k // 8, pl.ds((k % 8) * 16, 16)] = z16
        return 0

    lax.fori_loop(0, K * 8, zb, 0)
    _zero_acc_slice(ones_v, acc_sh, s)

    def ob(k, _):
        ones_v[k // 8, pl.ds((k % 8) * 16, 16)] = one16
        return 0

    lax.fori_loop(0, K * 8, ob, 0)
    pltpu.sync_copy(dst_hbm.at[pl.ds(wid * DCH, DCH)], dst_all)
    plsc.subcore_barrier()

    def grp(g, _):
        for b in range(DG):
            i = DG * g + b
            pltpu.async_copy(ones_v, acc_sh.at[dst_all.at[i]], sem, add=True)
        for b in range(DG):
            i = DG * g + b
            pltpu.make_async_copy(ones_v, acc_sh.at[dst_all.at[i]],
                                  sem).wait()
        return 0

    lax.fori_loop(0, DCH // DG, grp, 0)
    plsc.subcore_barrier()

    @pl.when(c == 0)
    def _():
        pltpu.sync_copy(acc_sh.at[pl.ds(s * RPT, RPT)],
                        out0.at[pl.ds(s * RPT, RPT)])

    @pl.when(c == 1)
    def _():
        pltpu.sync_copy(acc_sh.at[pl.ds(s * RPT, RPT)],
                        out1.at[pl.ds(s * RPT, RPT)])


_deg_kernel = functools.partial(
    pl.kernel, _deg_body, mesh=_MESH,
    out_type=[jax.ShapeDtypeStruct((ACC, 128), jnp.float32),
              jax.ShapeDtypeStruct((ACC, 128), jnp.float32)],
    scratch_types=[
        pltpu.VMEM((DCH, K), jnp.int32),
        pltpu.VMEM((K, 128), jnp.float32),
        pltpu.VMEM_SHARED((ACC, 128), jnp.float32),
        pltpu.SemaphoreType.DMA,
    ],
)()


def _scat_body(t_hbm, src_hbm, dst_hbm, out0, out1,
               src0, dst0, rows0, src1, dst1, rows1, src2, dst2, rows2,
               acc_sh, semg0, semg1, semg2, sems0, sems1, sems2):
    c = lax.axis_index("c")
    s = lax.axis_index("s")
    wid = s * NC + c
    z16 = jnp.zeros((16,), jnp.float32)

    def zb(k, _):
        rows0[k // 8, pl.ds((k % 8) * 16, 16)] = z16
        return 0

    lax.fori_loop(0, K * 8, zb, 0)
    _zero_acc_slice(rows0, acc_sh, s)
    plsc.subcore_barrier()

    base0 = wid * EPT
    sets = ((src0, dst0, rows0, semg0, sems0),
            (src1, dst1, rows1, semg1, sems1),
            (src2, dst2, rows2, semg2, sems2))

    def grp(g, _):
        i0 = G * g
        gh = []
        for b in range(G):
            st = sets[b]
            base = base0 + (i0 + b) * K
            pltpu.sync_copy(src_hbm.at[pl.ds(base, K)], st[0])
            pltpu.sync_copy(dst_hbm.at[pl.ds(base, K)], st[1])
            gh.append(pltpu.async_copy(t_hbm.at[st[0]], st[2], st[3]))
        sh = []
        for b in range(G):
            st = sets[b]
            gh[b].wait()
            sh.append(pltpu.async_copy(st[2], acc_sh.at[st[1]], st[4],
                                       add=True))
        for b in range(G):
            sh[b].wait()
        return 0

    lax.fori_loop(0, CHUNKS // G, grp, 0)
    plsc.subcore_barrier()

    @pl.when(c == 0)
    def _():
        pltpu.sync_copy(acc_sh.at[pl.ds(s * RPT, RPT)],
                        out0.at[pl.ds(s * RPT, RPT)])

    @pl.when(c == 1)
    def _():
        pltpu.sync_copy(acc_sh.at[pl.ds(s * RPT, RPT)],
                        out1.at[pl.ds(s * RPT, RPT)])


_scat_kernel = functools.partial(
    pl.kernel, _scat_body, mesh=_MESH,
    out_type=[jax.ShapeDtypeStruct((ACC, 128), jnp.float32),
              jax.ShapeDtypeStruct((ACC, 128), jnp.float32)],
    scratch_types=(
        [pltpu.VMEM((K,), jnp.int32), pltpu.VMEM((K,), jnp.int32),
         pltpu.VMEM((K, 128), jnp.float32)] * 3 +
        [pltpu.VMEM_SHARED((ACC, 128), jnp.float32)] +
        [pltpu.SemaphoreType.DMA] * 6
    ),
)()


_SPEC_FULL = pl.BlockSpec((N, 128), lambda: (0, 0))
_SPEC_COL = pl.BlockSpec((N, 1), lambda: (0, 0))
_SPEC_W = pl.BlockSpec((128, 128), lambda: (0, 0))
_SPEC_B = pl.BlockSpec((1, 128), lambda: (0, 0))


def _dinv(da_ref, db_ref):
    return lax.rsqrt(da_ref[...] + db_ref[...] + 1.0)


def _b1_body(x_ref, w_ref, da_ref, db_ref, o_ref):
    dinv = _dinv(da_ref, db_ref)
    o_ref[...] = jnp.dot(x_ref[...], w_ref[...],
                         preferred_element_type=jnp.float32) * dinv


def _b2_body(sa_ref, sb_ref, t_ref, da_ref, db_ref, b_ref, w_ref, o_ref):
    dinv = _dinv(da_ref, db_ref)
    agg = (sa_ref[...] + sb_ref[...] + t_ref[...]) * dinv + b_ref[...]
    h = jnp.maximum(agg, 0.0)
    o_ref[...] = jnp.dot(h, w_ref[...],
                         preferred_element_type=jnp.float32) * dinv


def _b3_body(sa_ref, sb_ref, t_ref, da_ref, db_ref, b_ref, o_ref):
    dinv = _dinv(da_ref, db_ref)
    o_ref[...] = (sa_ref[...] + sb_ref[...] + t_ref[...]) * dinv + b_ref[...]


def kernel(x, edge_index, W1, b1, W2, b2):
    src = edge_index[0].astype(jnp.int32)
    dst = edge_index[1].astype(jnp.int32)
    spad = jnp.zeros((EPAD - E,), jnp.int32)
    dpad = jnp.full((EPAD - E,), PADROW, jnp.int32)
    src_p = jnp.concatenate([src, spad])
    dst_p = jnp.concatenate([dst, dpad])
    dst2 = jnp.concatenate(
        [dst, jnp.full((DEPAD - E,), PADROW, jnp.int32)]).reshape(
            DEPAD // K, K)
    b1r = b1.reshape(1, 128)
    b2r = b2.reshape(1, 128)

    d0, d1 = _deg_kernel(dst2)
    d0c = d0[:N, :1]
    d1c = d1[:N, :1]

    f32 = jnp.float32
    t1 = pl.pallas_call(
        _b1_body, out_shape=jax.ShapeDtypeStruct((N, 128), f32),
        in_specs=[_SPEC_FULL, _SPEC_W, _SPEC_COL, _SPEC_COL],
        out_specs=_SPEC_FULL,
    )(x, W1, d0c, d1c)

    s1a, s1b = _scat_kernel(t1, src_p, dst_p)

    t2 = pl.pallas_call(
        _b2_body, out_shape=jax.ShapeDtypeStruct((N, 128), f32),
        in_specs=[_SPEC_FULL, _SPEC_FULL, _SPEC_FULL, _SPEC_COL, _SPEC_COL,
                  _SPEC_B, _SPEC_W],
        out_specs=_SPEC_FULL,
    )(s1a[:N], s1b[:N], t1, d0c, d1c, b1r, W2)

    s2a, s2b = _scat_kernel(t2, src_p, dst_p)

    z = pl.pallas_call(
        _b3_body, out_shape=jax.ShapeDtypeStruct((N, 128), f32),
        in_specs=[_SPEC_FULL, _SPEC_FULL, _SPEC_FULL, _SPEC_COL, _SPEC_COL,
                  _SPEC_B],
        out_specs=_SPEC_FULL,
    )(s2a[:N], s2b[:N], t2, d0c, d1c, b2r)

    return z


# speedup vs baseline: 1.1455x; 1.1455x over previous
import functools

import jax
import jax.numpy as jnp
from jax import lax
from jax.experimental import pallas as pl
from jax.experimental.pallas import tpu as pltpu
from jax.experimental.pallas import tpu_sc as plsc

N = 10000
D = 128
E = 320000
NPAD = 10240
NC = 2
NS = 16
NW = NC * NS
K = 128
EPAD = 327680
EPT = EPAD // NW
CHUNKS = EPT // K
RPT = NPAD // NS
PADROW = 10200
DG = 8

_MESH = plsc.VectorSubcoreMesh(core_axis_name="c", subcore_axis_name="s")


def _deg_body(dst_hbm, out0, out1, dst_all, ones_v, acc_sh, sem):
    c = lax.axis_index("c")
    s = lax.axis_index("s")
    wid = s * NC + c
    z16 = jnp.zeros((16,), jnp.float32)
    one16 = jnp.ones((16,), jnp.float32)

    def zb(k, _):
        ones_v[k // 8, pl.ds((k % 8) * 16, 16)] = z16
        return 0

    lax.fori_loop(0, K * 8, zb, 0)
    for b in range(RPT // 128):
        pltpu.sync_copy(ones_v, acc_sh.at[pl.ds(s * RPT + b * 128, 128)])

    def ob(k, _):
        ones_v[k // 8, pl.ds((k % 8) * 16, 16)] = one16
        return 0

    lax.fori_loop(0, K * 8, ob, 0)
    pltpu.sync_copy(dst_hbm.at[pl.ds(wid * CHUNKS, CHUNKS)], dst_all)
    plsc.subcore_barrier()

    def grp(g, _):
        for b in range(DG):
            i = DG * g + b
            pltpu.async_copy(ones_v, acc_sh.at[dst_all.at[i]], sem, add=True)
        for b in range(DG):
            i = DG * g + b
            pltpu.make_async_copy(ones_v, acc_sh.at[dst_all.at[i]],
                                  sem).wait()
        return 0

    lax.fori_loop(0, CHUNKS // DG, grp, 0)
    plsc.subcore_barrier()

    @pl.when(c == 0)
    def _():
        pltpu.sync_copy(acc_sh.at[pl.ds(s * RPT, RPT)],
                        out0.at[pl.ds(s * RPT, RPT)])

    @pl.when(c == 1)
    def _():
        pltpu.sync_copy(acc_sh.at[pl.ds(s * RPT, RPT)],
                        out1.at[pl.ds(s * RPT, RPT)])


_deg_kernel = functools.partial(
    pl.kernel, _deg_body, mesh=_MESH,
    out_type=[jax.ShapeDtypeStruct((NPAD, 128), jnp.float32),
              jax.ShapeDtypeStruct((NPAD, 128), jnp.float32)],
    scratch_types=[
        pltpu.VMEM((CHUNKS, K), jnp.int32),
        pltpu.VMEM((K, 128), jnp.float32),
        pltpu.VMEM_SHARED((NPAD, 128), jnp.float32),
        pltpu.SemaphoreType.DMA,
    ],
)()


def _scat_body(t_hbm, src_hbm, dst_hbm, out0, out1,
               src_v, dst_v, rows_v, zbuf, acc_sh, sem):
    c = lax.axis_index("c")
    s = lax.axis_index("s")
    wid = s * NC + c
    z16 = jnp.zeros((16,), jnp.float32)

    def zb(k, _):
        zbuf[k // 8, pl.ds((k % 8) * 16, 16)] = z16
        return 0

    lax.fori_loop(0, K * 8, zb, 0)
    for b in range(RPT // 128):
        pltpu.sync_copy(zbuf, acc_sh.at[pl.ds(s * RPT + b * 128, 128)])
    plsc.subcore_barrier()

    base0 = wid * EPT

    def chunk(i, _):
        base = base0 + i * K
        pltpu.sync_copy(src_hbm.at[pl.ds(base, K)], src_v)
        pltpu.sync_copy(dst_hbm.at[pl.ds(base, K)], dst_v)
        pltpu.async_copy(t_hbm.at[src_v], rows_v, sem).wait()
        pltpu.sync_copy(rows_v, acc_sh.at[dst_v], add=True)
        return 0

    lax.fori_loop(0, CHUNKS, chunk, 0)
    plsc.subcore_barrier()

    @pl.when(c == 0)
    def _():
        pltpu.sync_copy(acc_sh.at[pl.ds(s * RPT, RPT)],
                        out0.at[pl.ds(s * RPT, RPT)])

    @pl.when(c == 1)
    def _():
        pltpu.sync_copy(acc_sh.at[pl.ds(s * RPT, RPT)],
                        out1.at[pl.ds(s * RPT, RPT)])


_scat_kernel = functools.partial(
    pl.kernel, _scat_body, mesh=_MESH,
    out_type=[jax.ShapeDtypeStruct((NPAD, 128), jnp.float32),
              jax.ShapeDtypeStruct((NPAD, 128), jnp.float32)],
    scratch_types=[
        pltpu.VMEM((K,), jnp.int32),
        pltpu.VMEM((K,), jnp.int32),
        pltpu.VMEM((K, 128), jnp.float32),
        pltpu.VMEM((128, 128), jnp.float32),
        pltpu.VMEM_SHARED((NPAD, 128), jnp.float32),
        pltpu.SemaphoreType.DMA,
    ],
)()


_SPEC_FULL = pl.BlockSpec((NPAD, 128), lambda: (0, 0))
_SPEC_COL = pl.BlockSpec((NPAD, 1), lambda: (0, 0))
_SPEC_W = pl.BlockSpec((128, 128), lambda: (0, 0))
_SPEC_B = pl.BlockSpec((1, 128), lambda: (0, 0))


def _dinv(da_ref, db_ref):
    return lax.rsqrt(da_ref[...] + db_ref[...] + 1.0)


def _b1_body(x_ref, w_ref, da_ref, db_ref, o_ref):
    dinv = _dinv(da_ref, db_ref)
    o_ref[...] = jnp.dot(x_ref[...], w_ref[...],
                         preferred_element_type=jnp.float32) * dinv


def _b2_body(sa_ref, sb_ref, t_ref, da_ref, db_ref, b_ref, w_ref, o_ref):
    dinv = _dinv(da_ref, db_ref)
    agg = (sa_ref[...] + sb_ref[...] + t_ref[...]) * dinv + b_ref[...]
    h = jnp.maximum(agg, 0.0)
    t2 = jnp.dot(h, w_ref[...], preferred_element_type=jnp.float32) * dinv
    rows = lax.broadcasted_iota(jnp.int32, (NPAD, 128), 0)
    o_ref[...] = jnp.where(rows < N, t2, 0.0)


def _b3_body(sa_ref, sb_ref, t_ref, da_ref, db_ref, b_ref, o_ref):
    dinv = _dinv(da_ref, db_ref)
    o_ref[...] = (sa_ref[...] + sb_ref[...] + t_ref[...]) * dinv + b_ref[...]


def kernel(x, edge_index, W1, b1, W2, b2):
    src = edge_index[0].astype(jnp.int32)
    dst = edge_index[1].astype(jnp.int32)
    pad = jnp.full((EPAD - E,), PADROW, jnp.int32)
    src_p = jnp.concatenate([src, pad])
    dst_p = jnp.concatenate([dst, pad])
    dst2 = dst_p.reshape(EPAD // K, K)
    x_pad = jnp.pad(x, ((0, NPAD - N), (0, 0)))
    b1r = b1.reshape(1, 128)
    b2r = b2.reshape(1, 128)

    d0, d1 = _deg_kernel(dst2)
    d0c = d0[:, :1]
    d1c = d1[:, :1]

    f32 = jnp.float32
    t1 = pl.pallas_call(
        _b1_body, out_shape=jax.ShapeDtypeStruct((NPAD, 128), f32),
        in_specs=[_SPEC_FULL, _SPEC_W, _SPEC_COL, _SPEC_COL],
        out_specs=_SPEC_FULL,
    )(x_pad, W1, d0c, d1c)

    s1a, s1b = _scat_kernel(t1, src_p, dst_p)

    t2 = pl.pallas_call(
        _b2_body, out_shape=jax.ShapeDtypeStruct((NPAD, 128), f32),
        in_specs=[_SPEC_FULL, _SPEC_FULL, _SPEC_FULL, _SPEC_COL, _SPEC_COL,
                  _SPEC_B, _SPEC_W],
        out_specs=_SPEC_FULL,
    )(s1a, s1b, t1, d0c, d1c, b1r, W2)

    s2a, s2b = _scat_kernel(t2, src_p, dst_p)

    z = pl.pallas_call(
        _b3_body, out_shape=jax.ShapeDtypeStruct((NPAD, 128), f32),
        in_specs=[_SPEC_FULL, _SPEC_FULL, _SPEC_FULL, _SPEC_COL, _SPEC_COL,
                  _SPEC_B],
        out_specs=_SPEC_FULL,
    )(s2a, s2b, t2, d0c, d1c, b2r)

    return z[:N]
